# Initial kernel scaffold; baseline (speedup 1.0000x reference)
import jax
import jax.numpy as jnp
from jax.experimental import pallas as pl


def kernel(x, edge_index, edge_attr, W1, b1, W2, b2, W3, b3):
    raise NotImplementedError("write your pallas kernel here")


# speedup vs baseline: 3.7049x; 3.7049x over previous
import jax
import jax.numpy as jnp
from jax import lax
from jax.experimental import pallas as pl
from jax.experimental.pallas import tpu as pltpu
from jax.experimental.pallas import tpu_sc as plsc

N_NODES = 10000
N_EDGES = 320000
DIM = 128

NUM_CORES = 2
NUM_SUBCORES = 16
EDGES_PER_CORE = N_EDGES // NUM_CORES
EDGES_PER_SUBCORE = EDGES_PER_CORE // NUM_SUBCORES
CHUNK = 80
NUM_CHUNKS = EDGES_PER_SUBCORE // CHUNK
ROWS_A = 632
ROWS_B = N_NODES - (NUM_SUBCORES - 1) * ROWS_A


def _sc_edge_aggregate(x, src, dst, edge_attr):
    mesh = plsc.VectorSubcoreMesh(
        core_axis_name="c", subcore_axis_name="s", num_cores=NUM_CORES
    )

    @pl.kernel(
        out_type=jax.ShapeDtypeStruct((NUM_CORES, N_NODES, DIM), jnp.float32),
        mesh=mesh,
        scratch_types=[
            pltpu.VMEM_SHARED((N_NODES, DIM), jnp.float32),
            pltpu.VMEM((CHUNK, DIM), jnp.float32),
            pltpu.VMEM((CHUNK, DIM), jnp.float32),
            pltpu.VMEM((CHUNK,), jnp.int32),
            pltpu.VMEM((CHUNK,), jnp.int32),
            pltpu.SemaphoreType.DMA,
            pltpu.SemaphoreType.DMA,
        ],
    )
    def edge_kernel(x_hbm, src_hbm, dst_hbm, ea_hbm, out_hbm,
                    acc, rows, ea, sidx, didx, sem1, sem2):
        cid = lax.axis_index("c")
        sid = lax.axis_index("s")

        r0 = pl.multiple_of(sid * ROWS_A, 8)

        @pl.when(sid < NUM_SUBCORES - 1)
        def _init_main():
            pltpu.sync_copy(x_hbm.at[pl.ds(r0, ROWS_A)],
                            acc.at[pl.ds(r0, ROWS_A)])

        @pl.when(sid == NUM_SUBCORES - 1)
        def _init_tail():
            t0 = (NUM_SUBCORES - 1) * ROWS_A
            pltpu.sync_copy(x_hbm.at[pl.ds(t0, ROWS_B)],
                            acc.at[pl.ds(t0, ROWS_B)])

        plsc.subcore_barrier()

        base0 = cid * EDGES_PER_CORE + sid * EDGES_PER_SUBCORE

        @pl.loop(0, NUM_CHUNKS)
        def _chunk(ci):
            base = pl.multiple_of(base0 + ci * CHUNK, 8)
            pltpu.sync_copy(src_hbm.at[pl.ds(base, CHUNK)], sidx)
            pltpu.sync_copy(dst_hbm.at[pl.ds(base, CHUNK)], didx)
            gather = pltpu.async_copy(x_hbm.at[sidx], rows, sem1)
            load_ea = pltpu.async_copy(ea_hbm.at[pl.ds(base, CHUNK)], ea, sem2)
            gather.wait()
            load_ea.wait()

            @pl.loop(0, CHUNK)
            def _edge(e):
                @pl.loop(0, DIM, step=16)
                def _vec(c):
                    slc = (pl.ds(e, 1), pl.ds(c, 16))
                    v = rows.at[slc][...] + ea.at[slc][...]
                    rows.at[slc][...] = jnp.maximum(v, 0.0)

            pltpu.sync_copy(rows, acc.at[didx], add=True)

        plsc.subcore_barrier()

        @pl.when(sid < NUM_SUBCORES - 1)
        def _out_main():
            pltpu.sync_copy(acc.at[pl.ds(r0, ROWS_A)],
                            out_hbm.at[cid, pl.ds(r0, ROWS_A)])

        @pl.when(sid == NUM_SUBCORES - 1)
        def _out_tail():
            t0 = (NUM_SUBCORES - 1) * ROWS_A
            pltpu.sync_copy(acc.at[pl.ds(t0, ROWS_B)],
                            out_hbm.at[cid, pl.ds(t0, ROWS_B)])

    return edge_kernel(x, src, dst, edge_attr)


BLK = 1000


def _mlp_body(x_ref, a0_ref, a1_ref, w1_ref, b1_ref, w2_ref, b2_ref,
              w3_ref, b3_ref, o_ref):
    dn = (((1,), (0,)), ((), ()))
    h = a0_ref[...] + a1_ref[...] - x_ref[...]
    h = lax.dot_general(h, w1_ref[...], dn, precision=lax.Precision.HIGHEST,
                        preferred_element_type=jnp.float32)
    h = jnp.maximum(h + b1_ref[...], 0.0)
    h = lax.dot_general(h, w2_ref[...], dn, precision=lax.Precision.HIGHEST,
                        preferred_element_type=jnp.float32)
    h = jnp.maximum(h + b2_ref[...], 0.0)
    h = lax.dot_general(h, w3_ref[...], dn, precision=lax.Precision.HIGHEST,
                        preferred_element_type=jnp.float32)
    o_ref[...] = h + b3_ref[...]


def _tc_mlp(x, a0, a1, W1, b1, W2, b2, W3, b3):
    row_spec = pl.BlockSpec((BLK, DIM), lambda i: (i, 0))
    w_spec = pl.BlockSpec((DIM, DIM), lambda i: (0, 0))
    b_spec = pl.BlockSpec((1, DIM), lambda i: (0, 0))
    return pl.pallas_call(
        _mlp_body,
        grid=(N_NODES // BLK,),
        in_specs=[row_spec, row_spec, row_spec,
                  w_spec, b_spec, w_spec, b_spec, w_spec, b_spec],
        out_specs=row_spec,
        out_shape=jax.ShapeDtypeStruct((N_NODES, DIM), jnp.float32),
    )(x, a0, a1, W1, b1.reshape(1, DIM), W2, b2.reshape(1, DIM),
      W3, b3.reshape(1, DIM))


def kernel(x, edge_index, edge_attr, W1, b1, W2, b2, W3, b3):
    src = edge_index[0]
    dst = edge_index[1]
    acc = _sc_edge_aggregate(x, src, dst, edge_attr)
    return _tc_mlp(x, acc[0], acc[1], W1, b1, W2, b2, W3, b3)


# speedup vs baseline: 6.8235x; 1.8417x over previous
import jax
import jax.numpy as jnp
from jax import lax
from jax.experimental import pallas as pl
from jax.experimental.pallas import tpu as pltpu
from jax.experimental.pallas import tpu_sc as plsc

N_NODES = 10000
N_EDGES = 320000
DIM = 128

NUM_CORES = 2
NUM_SUBCORES = 16
NUM_WORKERS = NUM_CORES * NUM_SUBCORES
EDGES_PER_SUBCORE = N_EDGES // NUM_WORKERS
CHUNK = 48
NUM_CHUNKS = EDGES_PER_SUBCORE // CHUNK
NUM_PAIRS = NUM_CHUNKS // 2
TAIL = EDGES_PER_SUBCORE - NUM_CHUNKS * CHUNK
NCOL = DIM // 16
ROWS_A = 632
ROWS_B = N_NODES - (NUM_SUBCORES - 1) * ROWS_A


def _relu_add(rows, ea, res, n_edges):
    @plsc.parallel_loop(0, n_edges, step=1, unroll=4)
    def _edge(e):
        for cg in range(NCOL):
            slc = (pl.ds(e, 1), pl.ds(cg * 16, 16))
            v = rows.at[slc][...] + ea.at[slc][...]
            res.at[slc][...] = jnp.maximum(v, 0.0)


def _sc_edge_aggregate(x, src, dst, edge_attr):
    mesh = plsc.VectorSubcoreMesh(
        core_axis_name="c", subcore_axis_name="s", num_cores=NUM_CORES
    )

    @pl.kernel(
        out_type=jax.ShapeDtypeStruct((NUM_CORES, N_NODES, DIM), jnp.float32),
        mesh=mesh,
        scratch_types=[
            pltpu.VMEM_SHARED((N_NODES, DIM), jnp.float32),
            pltpu.VMEM((EDGES_PER_SUBCORE,), jnp.int32),
            pltpu.VMEM((CHUNK, DIM), jnp.float32),
            pltpu.VMEM((CHUNK, DIM), jnp.float32),
            pltpu.VMEM((CHUNK, DIM), jnp.float32),
            pltpu.VMEM((CHUNK, DIM), jnp.float32),
            pltpu.VMEM((CHUNK, DIM), jnp.float32),
            pltpu.VMEM((CHUNK, DIM), jnp.float32),
            pltpu.VMEM((CHUNK,), jnp.int32),
            pltpu.VMEM((CHUNK,), jnp.int32),
            pltpu.VMEM((TAIL,), jnp.int32),
            pltpu.SemaphoreType.DMA,
            pltpu.SemaphoreType.DMA,
            pltpu.SemaphoreType.DMA,
            pltpu.SemaphoreType.DMA,
            pltpu.SemaphoreType.DMA,
            pltpu.SemaphoreType.DMA,
        ],
    )
    def edge_kernel(x_hbm, src_hbm, dst_hbm, ea_hbm, out_hbm,
                    acc, sidx, rows0, rows1, ea0, ea1, res0, res1,
                    didx0, didx1, didxt,
                    semg0, semg1, semd0, semd1, sems0, sems1):
        cid = lax.axis_index("c")
        sid = lax.axis_index("s")
        w = cid * NUM_SUBCORES + sid
        ebase = pl.multiple_of(w * EDGES_PER_SUBCORE, 8)

        pltpu.sync_copy(src_hbm.at[pl.ds(ebase, EDGES_PER_SUBCORE)], sidx)

        r0 = pl.multiple_of(sid * ROWS_A, 8)

        @pl.when(sid < NUM_SUBCORES - 1)
        def _init_main():
            pltpu.sync_copy(x_hbm.at[pl.ds(r0, ROWS_A)],
                            acc.at[pl.ds(r0, ROWS_A)])

        @pl.when(sid == NUM_SUBCORES - 1)
        def _init_tail():
            t0 = (NUM_SUBCORES - 1) * ROWS_A
            pltpu.sync_copy(x_hbm.at[pl.ds(t0, ROWS_B)],
                            acc.at[pl.ds(t0, ROWS_B)])

        plsc.subcore_barrier()

        bufs = ((rows0, ea0, res0, didx0, semg0, semd0, sems0),
                (rows1, ea1, res1, didx1, semg1, semd1, sems1))

        def start_fetch(c, rows, ea, semg):
            eoff = pl.multiple_of(ebase + c * CHUNK, 8)
            pltpu.async_copy(
                x_hbm.at[sidx.at[pl.ds(c * CHUNK, CHUNK)]], rows, semg)
            pltpu.async_copy(ea_hbm.at[pl.ds(eoff, CHUNK)], ea, semg)

        def start_didx(c, didx, semd):
            eoff = pl.multiple_of(ebase + c * CHUNK, 8)
            pltpu.async_copy(dst_hbm.at[pl.ds(eoff, CHUNK)], didx, semd)

        for b in range(2):
            start_fetch(b, bufs[b][0], bufs[b][1], bufs[b][4])
            start_didx(b, bufs[b][3], bufs[b][5])

        @pl.loop(0, NUM_PAIRS)
        def _pair(p):
            for b in range(2):
                rows, ea, res, didx, semg, semd, sems = bufs[b]
                c = 2 * p + b
                pltpu.make_async_copy(
                    x_hbm.at[sidx.at[pl.ds(c * CHUNK, CHUNK)]],
                    rows, semg).wait()
                eoff = pl.multiple_of(ebase + c * CHUNK, 8)
                pltpu.make_async_copy(
                    ea_hbm.at[pl.ds(eoff, CHUNK)], ea, semg).wait()

                @pl.when(p > 0)
                def _drain_and_refetch():
                    pltpu.make_async_copy(res, acc.at[didx], sems).wait()
                    start_didx(c, didx, semd)

                _relu_add(rows, ea, res, CHUNK)

                pltpu.make_async_copy(
                    dst_hbm.at[pl.ds(eoff, CHUNK)], didx, semd).wait()
                pltpu.async_copy(res, acc.at[didx], sems, add=True)

                @pl.when(p < NUM_PAIRS - 1)
                def _prefetch():
                    start_fetch(c + 2, rows, ea, semg)

        for b in range(2):
            rows, ea, res, didx, semg, semd, sems = bufs[b]
            pltpu.make_async_copy(res, acc.at[didx], sems).wait()

        toff = pl.multiple_of(ebase + NUM_CHUNKS * CHUNK, 8)
        tsl = pl.ds(0, TAIL)
        pltpu.sync_copy(
            x_hbm.at[sidx.at[pl.ds(NUM_CHUNKS * CHUNK, TAIL)]],
            rows0.at[tsl])
        pltpu.sync_copy(ea_hbm.at[pl.ds(toff, TAIL)], ea0.at[tsl])
        pltpu.sync_copy(dst_hbm.at[pl.ds(toff, TAIL)], didxt)
        _relu_add(rows0, ea0, res0, TAIL)
        pltpu.sync_copy(res0.at[tsl], acc.at[didxt], add=True)

        plsc.subcore_barrier()

        @pl.when(sid < NUM_SUBCORES - 1)
        def _out_main():
            pltpu.sync_copy(acc.at[pl.ds(r0, ROWS_A)],
                            out_hbm.at[cid, pl.ds(r0, ROWS_A)])

        @pl.when(sid == NUM_SUBCORES - 1)
        def _out_tail():
            t0 = (NUM_SUBCORES - 1) * ROWS_A
            pltpu.sync_copy(acc.at[pl.ds(t0, ROWS_B)],
                            out_hbm.at[cid, pl.ds(t0, ROWS_B)])

    return edge_kernel(x, src, dst, edge_attr)


BLK = 1000


def _mlp_body(x_ref, a0_ref, a1_ref, w1_ref, b1_ref, w2_ref, b2_ref,
              w3_ref, b3_ref, o_ref):
    dn = (((1,), (0,)), ((), ()))
    h = a0_ref[...] + a1_ref[...] - x_ref[...]
    h = lax.dot_general(h, w1_ref[...], dn, precision=lax.Precision.HIGHEST,
                        preferred_element_type=jnp.float32)
    h = jnp.maximum(h + b1_ref[...], 0.0)
    h = lax.dot_general(h, w2_ref[...], dn, precision=lax.Precision.HIGHEST,
                        preferred_element_type=jnp.float32)
    h = jnp.maximum(h + b2_ref[...], 0.0)
    h = lax.dot_general(h, w3_ref[...], dn, precision=lax.Precision.HIGHEST,
                        preferred_element_type=jnp.float32)
    o_ref[...] = h + b3_ref[...]


def _tc_mlp(x, a0, a1, W1, b1, W2, b2, W3, b3):
    row_spec = pl.BlockSpec((BLK, DIM), lambda i: (i, 0))
    w_spec = pl.BlockSpec((DIM, DIM), lambda i: (0, 0))
    b_spec = pl.BlockSpec((1, DIM), lambda i: (0, 0))
    return pl.pallas_call(
        _mlp_body,
        grid=(N_NODES // BLK,),
        in_specs=[row_spec, row_spec, row_spec,
                  w_spec, b_spec, w_spec, b_spec, w_spec, b_spec],
        out_specs=row_spec,
        out_shape=jax.ShapeDtypeStruct((N_NODES, DIM), jnp.float32),
    )(x, a0, a1, W1, b1.reshape(1, DIM), W2, b2.reshape(1, DIM),
      W3, b3.reshape(1, DIM))


def kernel(x, edge_index, edge_attr, W1, b1, W2, b2, W3, b3):
    src = edge_index[0]
    dst = edge_index[1]
    acc = _sc_edge_aggregate(x, src, dst, edge_attr)
    return _tc_mlp(x, acc[0], acc[1], W1, b1, W2, b2, W3, b3)


# speedup vs baseline: 6.8329x; 1.0014x over previous
import jax
import jax.numpy as jnp
from jax import lax
from jax.experimental import pallas as pl
from jax.experimental.pallas import tpu as pltpu
from jax.experimental.pallas import tpu_sc as plsc

N_NODES = 10000
N_EDGES = 320000
DIM = 128

NUM_CORES = 2
NUM_SUBCORES = 16
NUM_WORKERS = NUM_CORES * NUM_SUBCORES
EDGES_PER_SUBCORE = N_EDGES // NUM_WORKERS
CHUNK = 48
NUM_CHUNKS = EDGES_PER_SUBCORE // CHUNK
NUM_PAIRS = NUM_CHUNKS // 2
TAIL = EDGES_PER_SUBCORE - NUM_CHUNKS * CHUNK
NCOL = DIM // 16
ROWS_A = 632
ROWS_B = N_NODES - (NUM_SUBCORES - 1) * ROWS_A


def _relu_add(rows, ea, res, n_edges):
    @plsc.parallel_loop(0, n_edges, step=1, unroll=8)
    def _edge(e):
        for cg in range(NCOL):
            slc = (pl.ds(e, 1), pl.ds(cg * 16, 16))
            v = rows.at[slc][...] + ea.at[slc][...]
            res.at[slc][...] = jnp.maximum(v, 0.0)


def _sc_edge_aggregate(x, src, dst, edge_attr):
    mesh = plsc.VectorSubcoreMesh(
        core_axis_name="c", subcore_axis_name="s", num_cores=NUM_CORES
    )

    @pl.kernel(
        out_type=jax.ShapeDtypeStruct((NUM_CORES, N_NODES, DIM), jnp.float32),
        mesh=mesh,
        scratch_types=[
            pltpu.VMEM_SHARED((N_NODES, DIM), jnp.float32),
            pltpu.VMEM((EDGES_PER_SUBCORE,), jnp.int32),
            pltpu.VMEM((CHUNK, DIM), jnp.float32),
            pltpu.VMEM((CHUNK, DIM), jnp.float32),
            pltpu.VMEM((CHUNK, DIM), jnp.float32),
            pltpu.VMEM((CHUNK, DIM), jnp.float32),
            pltpu.VMEM((CHUNK, DIM), jnp.float32),
            pltpu.VMEM((CHUNK, DIM), jnp.float32),
            pltpu.VMEM((CHUNK,), jnp.int32),
            pltpu.VMEM((CHUNK,), jnp.int32),
            pltpu.VMEM((TAIL,), jnp.int32),
            pltpu.SemaphoreType.DMA,
            pltpu.SemaphoreType.DMA,
            pltpu.SemaphoreType.DMA,
            pltpu.SemaphoreType.DMA,
            pltpu.SemaphoreType.DMA,
            pltpu.SemaphoreType.DMA,
        ],
    )
    def edge_kernel(x_hbm, src_hbm, dst_hbm, ea_hbm, out_hbm,
                    acc, sidx, rows0, rows1, ea0, ea1, res0, res1,
                    didx0, didx1, didxt,
                    semg0, semg1, semd0, semd1, sems0, sems1):
        cid = lax.axis_index("c")
        sid = lax.axis_index("s")
        w = cid * NUM_SUBCORES + sid
        ebase = pl.multiple_of(w * EDGES_PER_SUBCORE, 8)

        pltpu.sync_copy(src_hbm.at[pl.ds(ebase, EDGES_PER_SUBCORE)], sidx)

        r0 = pl.multiple_of(sid * ROWS_A, 8)

        @pl.when(sid < NUM_SUBCORES - 1)
        def _init_main():
            pltpu.sync_copy(x_hbm.at[pl.ds(r0, ROWS_A)],
                            acc.at[pl.ds(r0, ROWS_A)])

        @pl.when(sid == NUM_SUBCORES - 1)
        def _init_tail():
            t0 = (NUM_SUBCORES - 1) * ROWS_A
            pltpu.sync_copy(x_hbm.at[pl.ds(t0, ROWS_B)],
                            acc.at[pl.ds(t0, ROWS_B)])

        plsc.subcore_barrier()

        bufs = ((rows0, ea0, res0, didx0, semg0, semd0, sems0),
                (rows1, ea1, res1, didx1, semg1, semd1, sems1))

        def start_fetch(c, rows, ea, semg):
            eoff = pl.multiple_of(ebase + c * CHUNK, 8)
            pltpu.async_copy(
                x_hbm.at[sidx.at[pl.ds(c * CHUNK, CHUNK)]], rows, semg)
            pltpu.async_copy(ea_hbm.at[pl.ds(eoff, CHUNK)], ea, semg)

        def start_didx(c, didx, semd):
            eoff = pl.multiple_of(ebase + c * CHUNK, 8)
            pltpu.async_copy(dst_hbm.at[pl.ds(eoff, CHUNK)], didx, semd)

        for b in range(2):
            start_fetch(b, bufs[b][0], bufs[b][1], bufs[b][4])
            start_didx(b, bufs[b][3], bufs[b][5])

        @pl.loop(0, NUM_PAIRS)
        def _pair(p):
            for b in range(2):
                rows, ea, res, didx, semg, semd, sems = bufs[b]
                c = 2 * p + b
                pltpu.make_async_copy(
                    x_hbm.at[sidx.at[pl.ds(c * CHUNK, CHUNK)]],
                    rows, semg).wait()
                eoff = pl.multiple_of(ebase + c * CHUNK, 8)
                pltpu.make_async_copy(
                    ea_hbm.at[pl.ds(eoff, CHUNK)], ea, semg).wait()

                @pl.when(p > 0)
                def _drain_and_refetch():
                    pltpu.make_async_copy(res, acc.at[didx], sems).wait()
                    start_didx(c, didx, semd)

                _relu_add(rows, ea, res, CHUNK)

                pltpu.make_async_copy(
                    dst_hbm.at[pl.ds(eoff, CHUNK)], didx, semd).wait()
                pltpu.async_copy(res, acc.at[didx], sems, add=True)

                @pl.when(p < NUM_PAIRS - 1)
                def _prefetch():
                    start_fetch(c + 2, rows, ea, semg)

        for b in range(2):
            rows, ea, res, didx, semg, semd, sems = bufs[b]
            pltpu.make_async_copy(res, acc.at[didx], sems).wait()

        toff = pl.multiple_of(ebase + NUM_CHUNKS * CHUNK, 8)
        tsl = pl.ds(0, TAIL)
        pltpu.sync_copy(
            x_hbm.at[sidx.at[pl.ds(NUM_CHUNKS * CHUNK, TAIL)]],
            rows0.at[tsl])
        pltpu.sync_copy(ea_hbm.at[pl.ds(toff, TAIL)], ea0.at[tsl])
        pltpu.sync_copy(dst_hbm.at[pl.ds(toff, TAIL)], didxt)
        _relu_add(rows0, ea0, res0, TAIL)
        pltpu.sync_copy(res0.at[tsl], acc.at[didxt], add=True)

        plsc.subcore_barrier()

        @pl.when(sid < NUM_SUBCORES - 1)
        def _out_main():
            pltpu.sync_copy(acc.at[pl.ds(r0, ROWS_A)],
                            out_hbm.at[cid, pl.ds(r0, ROWS_A)])

        @pl.when(sid == NUM_SUBCORES - 1)
        def _out_tail():
            t0 = (NUM_SUBCORES - 1) * ROWS_A
            pltpu.sync_copy(acc.at[pl.ds(t0, ROWS_B)],
                            out_hbm.at[cid, pl.ds(t0, ROWS_B)])

    return edge_kernel(x, src, dst, edge_attr)


BLK = 1000


def _mlp_body(x_ref, a_ref, w1_ref, b1_ref, w2_ref, b2_ref,
              w3_ref, b3_ref, o_ref):
    dn = (((1,), (0,)), ((), ()))
    h = a_ref[0] + a_ref[1] - x_ref[...]
    h = lax.dot_general(h, w1_ref[...], dn, precision=lax.Precision.HIGHEST,
                        preferred_element_type=jnp.float32)
    h = jnp.maximum(h + b1_ref[...], 0.0)
    h = lax.dot_general(h, w2_ref[...], dn, precision=lax.Precision.HIGHEST,
                        preferred_element_type=jnp.float32)
    h = jnp.maximum(h + b2_ref[...], 0.0)
    h = lax.dot_general(h, w3_ref[...], dn, precision=lax.Precision.HIGHEST,
                        preferred_element_type=jnp.float32)
    o_ref[...] = h + b3_ref[...]


def _tc_mlp(x, acc, W1, b1, W2, b2, W3, b3):
    row_spec = pl.BlockSpec((BLK, DIM), lambda i: (i, 0))
    acc_spec = pl.BlockSpec((NUM_CORES, BLK, DIM), lambda i: (0, i, 0))
    w_spec = pl.BlockSpec((DIM, DIM), lambda i: (0, 0))
    b_spec = pl.BlockSpec((1, DIM), lambda i: (0, 0))
    return pl.pallas_call(
        _mlp_body,
        grid=(N_NODES // BLK,),
        in_specs=[row_spec, acc_spec,
                  w_spec, b_spec, w_spec, b_spec, w_spec, b_spec],
        out_specs=row_spec,
        out_shape=jax.ShapeDtypeStruct((N_NODES, DIM), jnp.float32),
    )(x, acc, W1, b1.reshape(1, DIM), W2, b2.reshape(1, DIM),
      W3, b3.reshape(1, DIM))


def kernel(x, edge_index, edge_attr, W1, b1, W2, b2, W3, b3):
    src = edge_index[0]
    dst = edge_index[1]
    acc = _sc_edge_aggregate(x, src, dst, edge_attr)
    return _tc_mlp(x, acc, W1, b1, W2, b2, W3, b3)


# speedup vs baseline: 7.5989x; 1.1121x over previous
import jax
import jax.numpy as jnp
from jax import lax
from jax.experimental import pallas as pl
from jax.experimental.pallas import tpu as pltpu
from jax.experimental.pallas import tpu_sc as plsc

N_NODES = 10000
N_EDGES = 320000
DIM = 128

NUM_CORES = 2
NUM_SUBCORES = 16
NUM_WORKERS = NUM_CORES * NUM_SUBCORES
EDGES_PER_SUBCORE = N_EDGES // NUM_WORKERS
CHUNK = 48
NUM_CHUNKS = EDGES_PER_SUBCORE // CHUNK
NUM_PAIRS = NUM_CHUNKS // 2
TAIL = EDGES_PER_SUBCORE - NUM_CHUNKS * CHUNK
NCOL = DIM // 16
ROWS_A = 632
ROWS_B = N_NODES - (NUM_SUBCORES - 1) * ROWS_A


def _relu_add(rows, ea, res, n_edges):
    @plsc.parallel_loop(0, n_edges, step=1, unroll=8)
    def _edge(e):
        for cg in range(NCOL):
            slc = (pl.ds(e, 1), pl.ds(cg * 16, 16))
            v = rows.at[slc][...] + ea.at[slc][...]
            res.at[slc][...] = jnp.maximum(v, 0.0)


def _sc_edge_aggregate(x, src, dst, edge_attr):
    mesh = plsc.VectorSubcoreMesh(
        core_axis_name="c", subcore_axis_name="s", num_cores=NUM_CORES
    )

    @pl.kernel(
        out_type=jax.ShapeDtypeStruct((NUM_CORES, N_NODES, DIM), jnp.float32),
        mesh=mesh,
        scratch_types=[
            pltpu.VMEM_SHARED((N_NODES, DIM), jnp.float32),
            pltpu.VMEM((EDGES_PER_SUBCORE,), jnp.int32),
            pltpu.VMEM((CHUNK, DIM), jnp.float32),
            pltpu.VMEM((CHUNK, DIM), jnp.float32),
            pltpu.VMEM((CHUNK, DIM), jnp.float32),
            pltpu.VMEM((CHUNK, DIM), jnp.float32),
            pltpu.VMEM((CHUNK, DIM), jnp.float32),
            pltpu.VMEM((CHUNK, DIM), jnp.float32),
            pltpu.VMEM((CHUNK,), jnp.int32),
            pltpu.VMEM((CHUNK,), jnp.int32),
            pltpu.VMEM((TAIL,), jnp.int32),
            pltpu.SemaphoreType.DMA,
            pltpu.SemaphoreType.DMA,
            pltpu.SemaphoreType.DMA,
            pltpu.SemaphoreType.DMA,
            pltpu.SemaphoreType.DMA,
            pltpu.SemaphoreType.DMA,
        ],
    )
    def edge_kernel(x_hbm, src_hbm, dst_hbm, ea_hbm, out_hbm,
                    acc, sidx, rows0, rows1, ea0, ea1, res0, res1,
                    didx0, didx1, didxt,
                    semg0, semg1, semd0, semd1, sems0, sems1):
        cid = lax.axis_index("c")
        sid = lax.axis_index("s")
        w = cid * NUM_SUBCORES + sid
        ebase = pl.multiple_of(w * EDGES_PER_SUBCORE, 8)

        pltpu.sync_copy(src_hbm.at[pl.ds(ebase, EDGES_PER_SUBCORE)], sidx)

        r0 = pl.multiple_of(sid * ROWS_A, 8)

        @pl.when(sid < NUM_SUBCORES - 1)
        def _init_main():
            pltpu.sync_copy(x_hbm.at[pl.ds(r0, ROWS_A)],
                            acc.at[pl.ds(r0, ROWS_A)])

        @pl.when(sid == NUM_SUBCORES - 1)
        def _init_tail():
            t0 = (NUM_SUBCORES - 1) * ROWS_A
            pltpu.sync_copy(x_hbm.at[pl.ds(t0, ROWS_B)],
                            acc.at[pl.ds(t0, ROWS_B)])

        plsc.subcore_barrier()

        bufs = ((rows0, ea0, res0, didx0, semg0, semd0, sems0),
                (rows1, ea1, res1, didx1, semg1, semd1, sems1))

        def start_fetch(c, rows, ea, semg):
            eoff = pl.multiple_of(ebase + c * CHUNK, 8)
            pltpu.async_copy(
                x_hbm.at[sidx.at[pl.ds(c * CHUNK, CHUNK)]], rows, semg)
            pltpu.async_copy(ea_hbm.at[pl.ds(eoff, CHUNK)], ea, semg)

        def start_didx(c, didx, semd):
            eoff = pl.multiple_of(ebase + c * CHUNK, 8)
            pltpu.async_copy(dst_hbm.at[pl.ds(eoff, CHUNK)], didx, semd)

        for b in range(2):
            start_fetch(b, bufs[b][0], bufs[b][1], bufs[b][4])
            start_didx(b, bufs[b][3], bufs[b][5])

        @pl.loop(0, NUM_PAIRS)
        def _pair(p):
            for b in range(2):
                rows, ea, res, didx, semg, semd, sems = bufs[b]
                c = 2 * p + b
                pltpu.make_async_copy(
                    x_hbm.at[sidx.at[pl.ds(c * CHUNK, CHUNK)]],
                    rows, semg).wait()
                eoff = pl.multiple_of(ebase + c * CHUNK, 8)
                pltpu.make_async_copy(
                    ea_hbm.at[pl.ds(eoff, CHUNK)], ea, semg).wait()

                @pl.when(p > 0)
                def _drain_and_refetch():
                    pltpu.make_async_copy(res, acc.at[didx], sems).wait()
                    start_didx(c, didx, semd)

                _relu_add(rows, ea, res, CHUNK)

                pltpu.make_async_copy(
                    dst_hbm.at[pl.ds(eoff, CHUNK)], didx, semd).wait()
                pltpu.async_copy(res, acc.at[didx], sems, add=True)

                @pl.when(p < NUM_PAIRS - 1)
                def _prefetch():
                    start_fetch(c + 2, rows, ea, semg)

        for b in range(2):
            rows, ea, res, didx, semg, semd, sems = bufs[b]
            pltpu.make_async_copy(res, acc.at[didx], sems).wait()

        toff = pl.multiple_of(ebase + NUM_CHUNKS * CHUNK, 8)
        tsl = pl.ds(0, TAIL)
        pltpu.sync_copy(
            x_hbm.at[sidx.at[pl.ds(NUM_CHUNKS * CHUNK, TAIL)]],
            rows0.at[tsl])
        pltpu.sync_copy(ea_hbm.at[pl.ds(toff, TAIL)], ea0.at[tsl])
        pltpu.sync_copy(dst_hbm.at[pl.ds(toff, TAIL)], didxt)
        _relu_add(rows0, ea0, res0, TAIL)
        pltpu.sync_copy(res0.at[tsl], acc.at[didxt], add=True)

        plsc.subcore_barrier()

        @pl.when(sid < NUM_SUBCORES - 1)
        def _out_main():
            pltpu.sync_copy(acc.at[pl.ds(r0, ROWS_A)],
                            out_hbm.at[cid, pl.ds(r0, ROWS_A)])

        @pl.when(sid == NUM_SUBCORES - 1)
        def _out_tail():
            t0 = (NUM_SUBCORES - 1) * ROWS_A
            pltpu.sync_copy(acc.at[pl.ds(t0, ROWS_B)],
                            out_hbm.at[cid, pl.ds(t0, ROWS_B)])

    return edge_kernel(x, src, dst, edge_attr)


BLK = 1000


def _mlp_body(x_ref, a_ref, w1_ref, b1_ref, w2_ref, b2_ref,
              w3_ref, b3_ref, o_ref):
    dn = (((1,), (0,)), ((), ()))
    h = a_ref[0] + a_ref[1] - x_ref[...]
    h = lax.dot_general(h, w1_ref[...], dn, precision=lax.Precision.DEFAULT,
                        preferred_element_type=jnp.float32)
    h = jnp.maximum(h + b1_ref[...], 0.0)
    h = lax.dot_general(h, w2_ref[...], dn, precision=lax.Precision.DEFAULT,
                        preferred_element_type=jnp.float32)
    h = jnp.maximum(h + b2_ref[...], 0.0)
    h = lax.dot_general(h, w3_ref[...], dn, precision=lax.Precision.DEFAULT,
                        preferred_element_type=jnp.float32)
    o_ref[...] = h + b3_ref[...]


def _tc_mlp(x, acc, W1, b1, W2, b2, W3, b3):
    row_spec = pl.BlockSpec((BLK, DIM), lambda i: (i, 0))
    acc_spec = pl.BlockSpec((NUM_CORES, BLK, DIM), lambda i: (0, i, 0))
    w_spec = pl.BlockSpec((DIM, DIM), lambda i: (0, 0))
    b_spec = pl.BlockSpec((1, DIM), lambda i: (0, 0))
    return pl.pallas_call(
        _mlp_body,
        grid=(N_NODES // BLK,),
        in_specs=[row_spec, acc_spec,
                  w_spec, b_spec, w_spec, b_spec, w_spec, b_spec],
        out_specs=row_spec,
        out_shape=jax.ShapeDtypeStruct((N_NODES, DIM), jnp.float32),
    )(x, acc, W1, b1.reshape(1, DIM), W2, b2.reshape(1, DIM),
      W3, b3.reshape(1, DIM))


def kernel(x, edge_index, edge_attr, W1, b1, W2, b2, W3, b3):
    src = edge_index[0]
    dst = edge_index[1]
    acc = _sc_edge_aggregate(x, src, dst, edge_attr)
    return _tc_mlp(x, acc, W1, b1, W2, b2, W3, b3)


# speedup vs baseline: 8.1309x; 1.0700x over previous
import jax
import jax.numpy as jnp
from jax import lax
from jax.experimental import pallas as pl
from jax.experimental.pallas import tpu as pltpu
from jax.experimental.pallas import tpu_sc as plsc

N_NODES = 10000
N_EDGES = 320000
DIM = 128

NUM_CORES = 2
NUM_SUBCORES = 16
NUM_WORKERS = NUM_CORES * NUM_SUBCORES
CHUNK = 64
PAIR = 2 * CHUNK
BASE_PAIRS = 78
EXTRA_W = 28
NCOL = DIM // 16
ROWS_A = 632
ROWS_B = N_NODES - (NUM_SUBCORES - 1) * ROWS_A


def _relu_add(rows, ea, res):
    @plsc.parallel_loop(0, CHUNK, step=1, unroll=8)
    def _edge(e):
        for cg in range(NCOL):
            slc = (pl.ds(e, 1), pl.ds(cg * 16, 16))
            v = rows.at[slc][...] + ea.at[slc][...]
            res.at[slc][...] = jnp.maximum(v, 0.0)


def _sc_edge_aggregate(x, edge_index, edge_attr):
    mesh = plsc.VectorSubcoreMesh(
        core_axis_name="c", subcore_axis_name="s", num_cores=NUM_CORES
    )

    @pl.kernel(
        out_type=jax.ShapeDtypeStruct((NUM_CORES, N_NODES, DIM), jnp.float32),
        mesh=mesh,
        scratch_types=[
            pltpu.VMEM_SHARED((N_NODES, DIM), jnp.float32),
            pltpu.VMEM((CHUNK, DIM), jnp.float32),
            pltpu.VMEM((CHUNK, DIM), jnp.float32),
            pltpu.VMEM((CHUNK, DIM), jnp.float32),
            pltpu.VMEM((CHUNK, DIM), jnp.float32),
            pltpu.VMEM((CHUNK, DIM), jnp.float32),
            pltpu.VMEM((CHUNK, DIM), jnp.float32),
            pltpu.VMEM((PAIR,), jnp.int32),
            pltpu.VMEM((PAIR,), jnp.int32),
            pltpu.VMEM((PAIR,), jnp.int32),
            pltpu.VMEM((PAIR,), jnp.int32),
            pltpu.VMEM((CHUNK,), jnp.int32),
            pltpu.VMEM((CHUNK,), jnp.int32),
            pltpu.SemaphoreType.DMA,
            pltpu.SemaphoreType.DMA,
            pltpu.SemaphoreType.DMA,
            pltpu.SemaphoreType.DMA,
            pltpu.SemaphoreType.DMA,
            pltpu.SemaphoreType.DMA,
            pltpu.SemaphoreType.DMA,
            pltpu.SemaphoreType.DMA,
        ],
    )
    def edge_kernel(x_hbm, ei_hbm, ea_hbm, out_hbm,
                    acc, rows0, rows1, ea0, ea1, res0, res1,
                    sf0, sf1, df0, df1, dh0, dh1,
                    semg0, semg1, semis0, semis1, semid0, semid1,
                    sems0, sems1):
        cid = lax.axis_index("c")
        sid = lax.axis_index("s")
        w = cid * NUM_SUBCORES + sid
        cb = pl.multiple_of(
            BASE_PAIRS * PAIR * w + PAIR * jnp.maximum(w - EXTRA_W, 0), 128)
        nchunks = jnp.where(w >= EXTRA_W, 2 * BASE_PAIRS + 2, 2 * BASE_PAIRS)

        r0 = pl.multiple_of(sid * ROWS_A, 8)

        @pl.when(sid < NUM_SUBCORES - 1)
        def _init_main():
            pltpu.sync_copy(x_hbm.at[pl.ds(r0, ROWS_A)],
                            acc.at[pl.ds(r0, ROWS_A)])

        @pl.when(sid == NUM_SUBCORES - 1)
        def _init_tail():
            t0 = (NUM_SUBCORES - 1) * ROWS_A
            pltpu.sync_copy(x_hbm.at[pl.ds(t0, ROWS_B)],
                            acc.at[pl.ds(t0, ROWS_B)])

        plsc.subcore_barrier()

        data = ((rows0, ea0, res0, dh0, semg0, sems0),
                (rows1, ea1, res1, dh1, semg1, sems1))
        idxb = ((sf0, df0, semis0, semid0),
                (sf1, df1, semis1, semid1))

        def pair_off(k):
            return pl.multiple_of(cb + k * PAIR, 128)

        def fetch_pair(k, q):
            sf, df, semis, semid = idxb[q]
            off = pair_off(k)
            pltpu.async_copy(ei_hbm.at[0, pl.ds(off, PAIR)], sf, semis)
            pltpu.async_copy(ei_hbm.at[1, pl.ds(off, PAIR)], df, semid)

        def start_fetch(c, b, q):
            rows, ea = data[b][0], data[b][1]
            semg = data[b][4]
            sf = idxb[q][0]
            pltpu.async_copy(
                x_hbm.at[sf.at[pl.ds(CHUNK * b, CHUNK)]], rows, semg)
            eoff = pl.multiple_of(cb + c * CHUNK, 64)
            pltpu.async_copy(ea_hbm.at[pl.ds(eoff, CHUNK)], ea, semg)

        fetch_pair(0, 0)
        fetch_pair(1, 1)
        pltpu.make_async_copy(
            ei_hbm.at[0, pl.ds(pair_off(0), PAIR)], sf0, semis0).wait()
        start_fetch(0, 0, 0)
        start_fetch(1, 1, 0)

        def chunk_body(c, b, q):
            rows, ea, res, dh, semg, sems = data[b]
            sf, df, semis, semid = idxb[q]
            pltpu.make_async_copy(
                x_hbm.at[sf.at[pl.ds(CHUNK * b, CHUNK)]], rows, semg).wait()
            eoff = pl.multiple_of(cb + c * CHUNK, 64)
            pltpu.make_async_copy(
                ea_hbm.at[pl.ds(eoff, CHUNK)], ea, semg).wait()

            @pl.when(c >= 2)
            def _drain():
                pltpu.make_async_copy(res, acc.at[dh], sems).wait()

            if b == 0:
                pltpu.make_async_copy(
                    ei_hbm.at[1, pl.ds(pair_off(c // 2), PAIR)],
                    df, semid).wait()
            for i in range(CHUNK // 16):
                dh.at[pl.ds(16 * i, 16)][...] = (
                    df.at[pl.ds(CHUNK * b + 16 * i, 16)][...])

            _relu_add(rows, ea, res)

            pltpu.async_copy(res, acc.at[dh], sems, add=True)

            @pl.when(c + 2 < nchunks)
            def _prefetch():
                if b == 0:
                    pltpu.make_async_copy(
                        ei_hbm.at[0, pl.ds(pair_off(c // 2 + 1), PAIR)],
                        idxb[1 - q][0], idxb[1 - q][2]).wait()
                start_fetch(c + 2, b, 1 - q)

            if b == 1:
                @pl.when(c + 3 < nchunks)
                def _refetch():
                    fetch_pair(c // 2 + 2, q)

        @pl.loop(0, BASE_PAIRS // 2)
        def _super(t):
            c0 = 4 * t
            chunk_body(c0 + 0, 0, 0)
            chunk_body(c0 + 1, 1, 0)
            chunk_body(c0 + 2, 0, 1)
            chunk_body(c0 + 3, 1, 1)

        @pl.when(w >= EXTRA_W)
        def _tail():
            chunk_body(2 * BASE_PAIRS + 0, 0, 0)
            chunk_body(2 * BASE_PAIRS + 1, 1, 0)

        for b in range(2):
            res, dh, sems = data[b][2], data[b][3], data[b][5]
            pltpu.make_async_copy(res, acc.at[dh], sems).wait()

        plsc.subcore_barrier()

        @pl.when(sid < NUM_SUBCORES - 1)
        def _out_main():
            pltpu.sync_copy(acc.at[pl.ds(r0, ROWS_A)],
                            out_hbm.at[cid, pl.ds(r0, ROWS_A)])

        @pl.when(sid == NUM_SUBCORES - 1)
        def _out_tail():
            t0 = (NUM_SUBCORES - 1) * ROWS_A
            pltpu.sync_copy(acc.at[pl.ds(t0, ROWS_B)],
                            out_hbm.at[cid, pl.ds(t0, ROWS_B)])

    return edge_kernel(x, edge_index, edge_attr)


BLK = 1000


def _mlp_body(x_ref, a_ref, w1_ref, b1_ref, w2_ref, b2_ref,
              w3_ref, b3_ref, o_ref):
    dn = (((1,), (0,)), ((), ()))
    h = a_ref[0] + a_ref[1] - x_ref[...]
    h = lax.dot_general(h, w1_ref[...], dn,
                        preferred_element_type=jnp.float32)
    h = jnp.maximum(h + b1_ref[...], 0.0)
    h = lax.dot_general(h, w2_ref[...], dn,
                        preferred_element_type=jnp.float32)
    h = jnp.maximum(h + b2_ref[...], 0.0)
    h = lax.dot_general(h, w3_ref[...], dn,
                        preferred_element_type=jnp.float32)
    o_ref[...] = h + b3_ref[...]


def _tc_mlp(x, acc, W1, b1, W2, b2, W3, b3):
    row_spec = pl.BlockSpec((BLK, DIM), lambda i: (i, 0))
    acc_spec = pl.BlockSpec((NUM_CORES, BLK, DIM), lambda i: (0, i, 0))
    w_spec = pl.BlockSpec((DIM, DIM), lambda i: (0, 0))
    b_spec = pl.BlockSpec((1, DIM), lambda i: (0, 0))
    return pl.pallas_call(
        _mlp_body,
        grid=(N_NODES // BLK,),
        in_specs=[row_spec, acc_spec,
                  w_spec, b_spec, w_spec, b_spec, w_spec, b_spec],
        out_specs=row_spec,
        out_shape=jax.ShapeDtypeStruct((N_NODES, DIM), jnp.float32),
    )(x, acc, W1, b1.reshape(1, DIM), W2, b2.reshape(1, DIM),
      W3, b3.reshape(1, DIM))


def kernel(x, edge_index, edge_attr, W1, b1, W2, b2, W3, b3):
    acc = _sc_edge_aggregate(x, edge_index, edge_attr)
    return _tc_mlp(x, acc, W1, b1, W2, b2, W3, b3)


# speedup vs baseline: 8.1811x; 1.0062x over previous
import jax
import jax.numpy as jnp
from jax import lax
from jax.experimental import pallas as pl
from jax.experimental.pallas import tpu as pltpu
from jax.experimental.pallas import tpu_sc as plsc

N_NODES = 10000
N_EDGES = 320000
DIM = 128

NUM_CORES = 2
NUM_SUBCORES = 16
NUM_WORKERS = NUM_CORES * NUM_SUBCORES
CHUNK = 64
PAIR = 2 * CHUNK
BASE_PAIRS = 78
EXTRA_W = 28
NCOL = DIM // 16
ROWS_A = 632
ROWS_B = N_NODES - (NUM_SUBCORES - 1) * ROWS_A


def _relu_add(rows, ea, res):
    @plsc.parallel_loop(0, CHUNK, step=1, unroll=8)
    def _edge(e):
        for cg in range(NCOL):
            slc = (pl.ds(e, 1), pl.ds(cg * 16, 16))
            v = rows.at[slc][...] + ea.at[slc][...]
            res.at[slc][...] = jnp.maximum(v, 0.0)


def _sc_edge_aggregate(x, edge_index, edge_attr):
    mesh = plsc.VectorSubcoreMesh(
        core_axis_name="c", subcore_axis_name="s", num_cores=NUM_CORES
    )

    @pl.kernel(
        out_type=jax.ShapeDtypeStruct((NUM_CORES, N_NODES, DIM), jnp.float32),
        mesh=mesh,
        scratch_types=[
            pltpu.VMEM_SHARED((N_NODES, DIM), jnp.float32),
            pltpu.VMEM((CHUNK, DIM), jnp.float32),
            pltpu.VMEM((CHUNK, DIM), jnp.float32),
            pltpu.VMEM((CHUNK, DIM), jnp.float32),
            pltpu.VMEM((CHUNK, DIM), jnp.float32),
            pltpu.VMEM((CHUNK, DIM), jnp.float32),
            pltpu.VMEM((CHUNK, DIM), jnp.float32),
            pltpu.VMEM((PAIR,), jnp.int32),
            pltpu.VMEM((PAIR,), jnp.int32),
            pltpu.VMEM((PAIR,), jnp.int32),
            pltpu.VMEM((PAIR,), jnp.int32),
            pltpu.VMEM((CHUNK,), jnp.int32),
            pltpu.VMEM((CHUNK,), jnp.int32),
            pltpu.SemaphoreType.DMA,
            pltpu.SemaphoreType.DMA,
            pltpu.SemaphoreType.DMA,
            pltpu.SemaphoreType.DMA,
            pltpu.SemaphoreType.DMA,
            pltpu.SemaphoreType.DMA,
            pltpu.SemaphoreType.DMA,
            pltpu.SemaphoreType.DMA,
        ],
    )
    def edge_kernel(x_hbm, ei_hbm, ea_hbm, out_hbm,
                    acc, rows0, rows1, ea0, ea1, res0, res1,
                    sf0, sf1, df0, df1, dh0, dh1,
                    semg0, semg1, semis0, semis1, semid0, semid1,
                    sems0, sems1):
        cid = lax.axis_index("c")
        sid = lax.axis_index("s")
        w = cid * NUM_SUBCORES + sid
        cb = pl.multiple_of(
            BASE_PAIRS * PAIR * w + PAIR * jnp.maximum(w - EXTRA_W, 0), 128)
        nchunks = jnp.where(w >= EXTRA_W, 2 * BASE_PAIRS + 2, 2 * BASE_PAIRS)
        r0 = pl.multiple_of(sid * ROWS_A, 8)

        data = ((rows0, ea0, res0, dh0, semg0, sems0),
                (rows1, ea1, res1, dh1, semg1, sems1))
        idxb = ((sf0, df0, semis0, semid0),
                (sf1, df1, semis1, semid1))

        def pair_off(k):
            return pl.multiple_of(cb + k * PAIR, 128)

        def fetch_pair(k, q):
            sf, df, semis, semid = idxb[q]
            off = pair_off(k)
            pltpu.async_copy(ei_hbm.at[0, pl.ds(off, PAIR)], sf, semis)
            pltpu.async_copy(ei_hbm.at[1, pl.ds(off, PAIR)], df, semid)

        def start_fetch(c, b, q):
            rows, ea = data[b][0], data[b][1]
            semg = data[b][4]
            sf = idxb[q][0]
            pltpu.async_copy(
                x_hbm.at[sf.at[pl.ds(CHUNK * b, CHUNK)]], rows, semg)
            eoff = pl.multiple_of(cb + c * CHUNK, 64)
            pltpu.async_copy(ea_hbm.at[pl.ds(eoff, CHUNK)], ea, semg)

        fetch_pair(0, 0)
        fetch_pair(1, 1)
        pltpu.make_async_copy(
            ei_hbm.at[0, pl.ds(pair_off(0), PAIR)], sf0, semis0).wait()
        start_fetch(0, 0, 0)
        start_fetch(1, 1, 0)

        @pl.when(sid < NUM_SUBCORES - 1)
        def _init_main():
            pltpu.sync_copy(x_hbm.at[pl.ds(r0, ROWS_A)],
                            acc.at[pl.ds(r0, ROWS_A)])

        @pl.when(sid == NUM_SUBCORES - 1)
        def _init_tail():
            t0 = (NUM_SUBCORES - 1) * ROWS_A
            pltpu.sync_copy(x_hbm.at[pl.ds(t0, ROWS_B)],
                            acc.at[pl.ds(t0, ROWS_B)])

        plsc.subcore_barrier()

        def chunk_body(c, b, q):
            rows, ea, res, dh, semg, sems = data[b]
            sf, df, semis, semid = idxb[q]
            pltpu.make_async_copy(
                x_hbm.at[sf.at[pl.ds(CHUNK * b, CHUNK)]], rows, semg).wait()
            eoff = pl.multiple_of(cb + c * CHUNK, 64)
            pltpu.make_async_copy(
                ea_hbm.at[pl.ds(eoff, CHUNK)], ea, semg).wait()

            @pl.when(c >= 2)
            def _drain():
                pltpu.make_async_copy(res, acc.at[dh], sems).wait()

            if b == 0:
                pltpu.make_async_copy(
                    ei_hbm.at[1, pl.ds(pair_off(c // 2), PAIR)],
                    df, semid).wait()
            for i in range(CHUNK // 16):
                dh.at[pl.ds(16 * i, 16)][...] = (
                    df.at[pl.ds(CHUNK * b + 16 * i, 16)][...])

            _relu_add(rows, ea, res)

            pltpu.async_copy(res, acc.at[dh], sems, add=True)

            @pl.when(c + 2 < nchunks)
            def _prefetch():
                if b == 0:
                    pltpu.make_async_copy(
                        ei_hbm.at[0, pl.ds(pair_off(c // 2 + 1), PAIR)],
                        idxb[1 - q][0], idxb[1 - q][2]).wait()
                start_fetch(c + 2, b, 1 - q)

            if b == 1:
                @pl.when(c + 3 < nchunks)
                def _refetch():
                    fetch_pair(c // 2 + 2, q)

        @pl.loop(0, BASE_PAIRS // 2)
        def _super(t):
            c0 = 4 * t
            chunk_body(c0 + 0, 0, 0)
            chunk_body(c0 + 1, 1, 0)
            chunk_body(c0 + 2, 0, 1)
            chunk_body(c0 + 3, 1, 1)

        @pl.when(w >= EXTRA_W)
        def _tail():
            chunk_body(2 * BASE_PAIRS + 0, 0, 0)
            chunk_body(2 * BASE_PAIRS + 1, 1, 0)

        for b in range(2):
            res, dh, sems = data[b][2], data[b][3], data[b][5]
            pltpu.make_async_copy(res, acc.at[dh], sems).wait()

        plsc.subcore_barrier()

        @pl.when(sid < NUM_SUBCORES - 1)
        def _out_main():
            pltpu.sync_copy(acc.at[pl.ds(r0, ROWS_A)],
                            out_hbm.at[cid, pl.ds(r0, ROWS_A)])

        @pl.when(sid == NUM_SUBCORES - 1)
        def _out_tail():
            t0 = (NUM_SUBCORES - 1) * ROWS_A
            pltpu.sync_copy(acc.at[pl.ds(t0, ROWS_B)],
                            out_hbm.at[cid, pl.ds(t0, ROWS_B)])

    return edge_kernel(x, edge_index, edge_attr)


BLK = 1000


def _mlp_body(x_ref, a_ref, w1_ref, b1_ref, w2_ref, b2_ref,
              w3_ref, b3_ref, o_ref):
    dn = (((1,), (0,)), ((), ()))
    h = a_ref[0] + a_ref[1] - x_ref[...]
    h = lax.dot_general(h, w1_ref[...], dn,
                        preferred_element_type=jnp.float32)
    h = jnp.maximum(h + b1_ref[...], 0.0)
    h = lax.dot_general(h, w2_ref[...], dn,
                        preferred_element_type=jnp.float32)
    h = jnp.maximum(h + b2_ref[...], 0.0)
    h = lax.dot_general(h, w3_ref[...], dn,
                        preferred_element_type=jnp.float32)
    o_ref[...] = h + b3_ref[...]


def _tc_mlp(x, acc, W1, b1, W2, b2, W3, b3):
    row_spec = pl.BlockSpec((BLK, DIM), lambda i: (i, 0))
    acc_spec = pl.BlockSpec((NUM_CORES, BLK, DIM), lambda i: (0, i, 0))
    w_spec = pl.BlockSpec((DIM, DIM), lambda i: (0, 0))
    b_spec = pl.BlockSpec((1, DIM), lambda i: (0, 0))
    return pl.pallas_call(
        _mlp_body,
        grid=(N_NODES // BLK,),
        in_specs=[row_spec, acc_spec,
                  w_spec, b_spec, w_spec, b_spec, w_spec, b_spec],
        out_specs=row_spec,
        out_shape=jax.ShapeDtypeStruct((N_NODES, DIM), jnp.float32),
    )(x, acc, W1, b1.reshape(1, DIM), W2, b2.reshape(1, DIM),
      W3, b3.reshape(1, DIM))


def kernel(x, edge_index, edge_attr, W1, b1, W2, b2, W3, b3):
    acc = _sc_edge_aggregate(x, edge_index, edge_attr)
    return _tc_mlp(x, acc, W1, b1, W2, b2, W3, b3)
